# baseline (speedup 1.0000x reference)
import dataclasses
import functools

import jax
import jax.numpy as jnp
from jax import lax
from jax.experimental import pallas as pl
from jax.experimental.pallas import tpu as pltpu
from jax.experimental.pallas import tpu_sc as plsc

N = 10000
E = 320000
IN = 128
HID = 256
OUT = 64

NC = 2
NS = 16
L = 16
CHUNK = 128
CPT = 79
EDGES_PER_TILE = CPT * CHUNK
E_PAD = NC * NS * EDGES_PER_TILE
ROWS_PER_TILE = 640
NROWS = NS * ROWS_PER_TILE
ZCOPIES = ROWS_PER_TILE // CHUNK

_MESH = plsc.VectorSubcoreMesh(core_axis_name="c", subcore_axis_name="s")

_SC_PARAMS = pltpu.CompilerParams()
if "needs_layout_passes" in pltpu.CompilerParams.__dataclass_fields__:
    _SC_PARAMS = dataclasses.replace(_SC_PARAMS, needs_layout_passes=False)


def _sc_segsum_body(with_deg, x_hbm, src_hbm, dst_hbm, zf_hbm, zd_hbm,
                    s_out, d_out, src_a, dst_a, src_b, dst_b, rows_a, rows_b,
                    deg_v, acc_sh, sem_a, sem_b):
    c = lax.axis_index("c")
    s = lax.axis_index("s")
    r0 = s * ROWS_PER_TILE
    pltpu.sync_copy(zf_hbm, rows_a)
    for j in range(ZCOPIES):
        pltpu.sync_copy(rows_a, acc_sh.at[pl.ds(r0 + j * CHUNK, CHUNK)])
    if with_deg:
        pltpu.sync_copy(zd_hbm, deg_v)
    plsc.subcore_barrier()
    base = (c * NS + s) * EDGES_PER_TILE

    def load_idx(j, sbuf, dbuf):
        off = base + j * CHUNK
        pltpu.sync_copy(src_hbm.at[pl.ds(off, CHUNK)], sbuf)
        pltpu.sync_copy(dst_hbm.at[pl.ds(off, CHUNK)], dbuf)

    def deg_update(dbuf):
        if with_deg:
            ones = jnp.ones((L,), jnp.float32)
            for k in range(CHUNK // L):
                plsc.addupdate_scatter(deg_v, [dbuf[pl.ds(k * L, L)]], ones)

    def issue_gather(sbuf, rbuf, sem):
        h = CHUNK // 2
        pltpu.async_copy(x_hbm.at[sbuf.at[pl.ds(0, h)]], rbuf.at[pl.ds(0, h)], sem)
        pltpu.async_copy(x_hbm.at[sbuf.at[pl.ds(h, h)]], rbuf.at[pl.ds(h, h)], sem)

    def wait_gather(sbuf, rbuf, sem):
        pltpu.make_async_copy(x_hbm.at[sbuf], rbuf, sem).wait()

    load_idx(0, src_a, dst_a)
    issue_gather(src_a, rows_a, sem_a)

    def pair(it, carry):
        j = 2 * it
        load_idx(j + 1, src_b, dst_b)
        issue_gather(src_b, rows_b, sem_b)
        wait_gather(src_a, rows_a, sem_a)
        deg_update(dst_a)
        pltpu.sync_copy(rows_a, acc_sh.at[dst_a], add=True)
        load_idx(j + 2, src_a, dst_a)
        issue_gather(src_a, rows_a, sem_a)
        wait_gather(src_b, rows_b, sem_b)
        deg_update(dst_b)
        pltpu.sync_copy(rows_b, acc_sh.at[dst_b], add=True)
        return carry

    lax.fori_loop(0, (CPT - 1) // 2, pair, 0)
    wait_gather(src_a, rows_a, sem_a)
    deg_update(dst_a)
    pltpu.sync_copy(rows_a, acc_sh.at[dst_a], add=True)
    plsc.subcore_barrier()
    for j in range(ZCOPIES):
        pltpu.sync_copy(acc_sh.at[pl.ds(r0 + j * CHUNK, CHUNK)], rows_a)
        pltpu.sync_copy(rows_a, s_out.at[c, pl.ds(r0 + j * CHUNK, CHUNK)])
    if with_deg:
        pltpu.sync_copy(deg_v, d_out.at[c, s])


@functools.partial(
    pl.kernel,
    out_type=(jax.ShapeDtypeStruct((NC, NROWS, IN), jnp.float32),
              jax.ShapeDtypeStruct((NC, NS, NROWS), jnp.float32)),
    mesh=_MESH,
    compiler_params=_SC_PARAMS,
    scratch_types=[
        pltpu.VMEM((CHUNK,), jnp.int32),
        pltpu.VMEM((CHUNK,), jnp.int32),
        pltpu.VMEM((CHUNK,), jnp.int32),
        pltpu.VMEM((CHUNK,), jnp.int32),
        pltpu.VMEM((CHUNK, IN), jnp.float32),
        pltpu.VMEM((CHUNK, IN), jnp.float32),
        pltpu.VMEM((NROWS,), jnp.float32),
        pltpu.VMEM_SHARED((NROWS, IN), jnp.float32),
        pltpu.SemaphoreType.DMA,
        pltpu.SemaphoreType.DMA,
    ],
)
def _sc_layer1(x_hbm, src_hbm, dst_hbm, zf_hbm, zd_hbm,
               s_out, d_out, src_a, dst_a, src_b, dst_b, rows_a, rows_b,
               deg_v, acc_sh, sem_a, sem_b):
    _sc_segsum_body(True, x_hbm, src_hbm, dst_hbm, zf_hbm, zd_hbm,
                    s_out, d_out, src_a, dst_a, src_b, dst_b, rows_a, rows_b,
                    deg_v, acc_sh, sem_a, sem_b)


@functools.partial(
    pl.kernel,
    out_type=jax.ShapeDtypeStruct((NC, NROWS, IN), jnp.float32),
    mesh=_MESH,
    compiler_params=_SC_PARAMS,
    scratch_types=[
        pltpu.VMEM((CHUNK,), jnp.int32),
        pltpu.VMEM((CHUNK,), jnp.int32),
        pltpu.VMEM((CHUNK,), jnp.int32),
        pltpu.VMEM((CHUNK,), jnp.int32),
        pltpu.VMEM((CHUNK, IN), jnp.float32),
        pltpu.VMEM((CHUNK, IN), jnp.float32),
        pltpu.VMEM_SHARED((NROWS, IN), jnp.float32),
        pltpu.SemaphoreType.DMA,
        pltpu.SemaphoreType.DMA,
    ],
)
def _sc_layer2(p_hbm, src_hbm, dst_hbm, zf_hbm,
               s_out, src_a, dst_a, src_b, dst_b, rows_a, rows_b,
               acc_sh, sem_a, sem_b):
    _sc_segsum_body(False, p_hbm, src_hbm, dst_hbm, zf_hbm, None,
                    s_out, None, src_a, dst_a, src_b, dst_b, rows_a, rows_b,
                    None, acc_sh, sem_a, sem_b)


R = 1024


def _tc1_body(s1_ref, deg_ref, x_ref, w1l_ref, b1_ref, w1r_ref,
              w2l_ref, w2r_ref, pr_ref):
    sv = s1_ref[0] + s1_ref[1]
    deg = jnp.sum(deg_ref[...], axis=(0, 1))[:, None]
    agg = sv / jnp.maximum(deg, 1.0)
    h = jnp.dot(agg, w1l_ref[...], preferred_element_type=jnp.float32)
    h = h + b1_ref[...] + jnp.dot(x_ref[...], w1r_ref[...],
                                  preferred_element_type=jnp.float32)
    h = jnp.maximum(h, 0.0)
    p = jnp.dot(h, w2l_ref[...], preferred_element_type=jnp.float32)
    r = jnp.dot(h, w2r_ref[...], preferred_element_type=jnp.float32)
    pr_ref[...] = jnp.concatenate([p, r], axis=1)


_tc1 = pl.pallas_call(
    _tc1_body,
    grid=(NROWS // R,),
    in_specs=[
        pl.BlockSpec((NC, R, IN), lambda i: (0, i, 0)),
        pl.BlockSpec((NC, NS, R), lambda i: (0, 0, i)),
        pl.BlockSpec((R, IN), lambda i: (i, 0)),
        pl.BlockSpec((IN, HID), lambda i: (0, 0)),
        pl.BlockSpec((1, HID), lambda i: (0, 0)),
        pl.BlockSpec((IN, HID), lambda i: (0, 0)),
        pl.BlockSpec((HID, OUT), lambda i: (0, 0)),
        pl.BlockSpec((HID, OUT), lambda i: (0, 0)),
    ],
    out_specs=pl.BlockSpec((R, IN), lambda i: (i, 0)),
    out_shape=jax.ShapeDtypeStruct((NROWS, IN), jnp.float32),
)


def _tc2_body(s2_ref, deg_ref, pr_ref, b2_ref, o_ref):
    sv = s2_ref[0, :, 0:OUT] + s2_ref[1, :, 0:OUT]
    deg = jnp.sum(deg_ref[...], axis=(0, 1))[:, None]
    o = sv / jnp.maximum(deg, 1.0) + b2_ref[...] + pr_ref[:, OUT:IN]
    m = jnp.max(o, axis=1, keepdims=True)
    e = jnp.exp(o - m)
    o_ref[...] = (o - m) - jnp.log(jnp.sum(e, axis=1, keepdims=True))


_tc2 = pl.pallas_call(
    _tc2_body,
    grid=(NROWS // R,),
    in_specs=[
        pl.BlockSpec((NC, R, IN), lambda i: (0, i, 0)),
        pl.BlockSpec((NC, NS, R), lambda i: (0, 0, i)),
        pl.BlockSpec((R, IN), lambda i: (i, 0)),
        pl.BlockSpec((1, OUT), lambda i: (0, 0)),
    ],
    out_specs=pl.BlockSpec((R, OUT), lambda i: (i, 0)),
    out_shape=jax.ShapeDtypeStruct((NROWS, OUT), jnp.float32),
)


def kernel(x, edge_index, W1l, b1, W1r, W2l, b2, W2r):
    src = jnp.pad(edge_index[0], (0, E_PAD - E))
    dst = jnp.pad(edge_index[1], (0, E_PAD - E), constant_values=N)
    xp = jnp.pad(x, ((0, NROWS - N), (0, 0)))
    zf = jnp.zeros((CHUNK, IN), jnp.float32)
    zd = jnp.zeros((NROWS,), jnp.float32)
    s1, degp = _sc_layer1(xp, src, dst, zf, zd)
    pr = _tc1(s1, degp, xp, W1l.T, b1.reshape(1, HID), W1r.T,
              W2l.T, W2r.T)
    s2 = _sc_layer2(pr, src, dst, zf)
    return _tc2(s2, degp, pr, b2.reshape(1, OUT))[:N]


# speedup vs baseline: 1.0443x; 1.0443x over previous
import dataclasses
import functools

import jax
import jax.numpy as jnp
from jax import lax
from jax.experimental import pallas as pl
from jax.experimental.pallas import tpu as pltpu
from jax.experimental.pallas import tpu_sc as plsc

N = 10000
E = 320000
IN = 128
HID = 256
OUT = 64

NC = 2
NS = 16
L = 16
CHUNK = 128
CPT = 79
EDGES_PER_TILE = CPT * CHUNK
E_PAD = NC * NS * EDGES_PER_TILE
ROWS_PER_TILE = 640
NROWS = NS * ROWS_PER_TILE
ZCOPIES = ROWS_PER_TILE // CHUNK

_MESH = plsc.VectorSubcoreMesh(core_axis_name="c", subcore_axis_name="s")

_SC_PARAMS = pltpu.CompilerParams()
if "needs_layout_passes" in pltpu.CompilerParams.__dataclass_fields__:
    _SC_PARAMS = dataclasses.replace(_SC_PARAMS, needs_layout_passes=False)


def _sc_segsum_body(with_deg, x_hbm, src_hbm, dst_hbm, zf_hbm, zd_hbm,
                    s_out, d_out, src_a, dst_a, src_b, dst_b, rows_a, rows_b,
                    deg_v, acc_sh, sem_a, sem_b, isem_a, isem_b):
    c = lax.axis_index("c")
    s = lax.axis_index("s")
    r0 = s * ROWS_PER_TILE
    pltpu.sync_copy(zf_hbm, rows_a)
    for j in range(ZCOPIES):
        pltpu.sync_copy(rows_a, acc_sh.at[pl.ds(r0 + j * CHUNK, CHUNK)])
    if with_deg:
        pltpu.sync_copy(zd_hbm, deg_v)
    plsc.subcore_barrier()
    base = (c * NS + s) * EDGES_PER_TILE

    def load_idx(j, sbuf, dbuf):
        off = base + j * CHUNK
        pltpu.sync_copy(src_hbm.at[pl.ds(off, CHUNK)], sbuf)
        pltpu.sync_copy(dst_hbm.at[pl.ds(off, CHUNK)], dbuf)

    def issue_idx(j, sbuf, dbuf, isem):
        off = base + j * CHUNK
        pltpu.async_copy(src_hbm.at[pl.ds(off, CHUNK)], sbuf, isem)
        pltpu.async_copy(dst_hbm.at[pl.ds(off, CHUNK)], dbuf, isem)

    def wait_idx(sbuf, dbuf, isem):
        pltpu.make_async_copy(src_hbm.at[pl.ds(0, CHUNK)], sbuf, isem).wait()
        pltpu.make_async_copy(dst_hbm.at[pl.ds(0, CHUNK)], dbuf, isem).wait()

    def deg_update(dbuf):
        if with_deg:
            ones = jnp.ones((L,), jnp.float32)
            for k in range(CHUNK // L):
                plsc.addupdate_scatter(deg_v, [dbuf[pl.ds(k * L, L)]], ones)

    def issue_gather(sbuf, rbuf, sem):
        h = CHUNK // 2
        pltpu.async_copy(x_hbm.at[sbuf.at[pl.ds(0, h)]], rbuf.at[pl.ds(0, h)], sem)
        pltpu.async_copy(x_hbm.at[sbuf.at[pl.ds(h, h)]], rbuf.at[pl.ds(h, h)], sem)

    def wait_gather(sbuf, rbuf, sem):
        pltpu.make_async_copy(x_hbm.at[sbuf], rbuf, sem).wait()

    load_idx(0, src_a, dst_a)
    issue_gather(src_a, rows_a, sem_a)
    issue_idx(1, src_b, dst_b, isem_b)

    def step(k, sbufP, dbufP, rowsP, semP, sbufQ, dbufQ, rowsQ, semQ,
             isemP, isemQ):
        wait_idx(sbufQ, dbufQ, isemQ)
        issue_gather(sbufQ, rowsQ, semQ)
        wait_gather(sbufP, rowsP, semP)
        pltpu.sync_copy(rowsP, acc_sh.at[dbufP], add=True)
        deg_update(dbufP)

        @pl.when(k + 2 < CPT)
        def _():
            issue_idx(k + 2, sbufP, dbufP, isemP)

    def pair(it, carry):
        j = 2 * it
        step(j, src_a, dst_a, rows_a, sem_a, src_b, dst_b, rows_b, sem_b,
             isem_a, isem_b)
        step(j + 1, src_b, dst_b, rows_b, sem_b, src_a, dst_a, rows_a, sem_a,
             isem_b, isem_a)
        return carry

    lax.fori_loop(0, (CPT - 1) // 2, pair, 0)
    wait_gather(src_a, rows_a, sem_a)
    pltpu.sync_copy(rows_a, acc_sh.at[dst_a], add=True)
    deg_update(dst_a)
    plsc.subcore_barrier()
    for j in range(ZCOPIES):
        pltpu.sync_copy(acc_sh.at[pl.ds(r0 + j * CHUNK, CHUNK)], rows_a)
        pltpu.sync_copy(rows_a, s_out.at[c, pl.ds(r0 + j * CHUNK, CHUNK)])
    if with_deg:
        pltpu.sync_copy(deg_v, d_out.at[c, s])


@functools.partial(
    pl.kernel,
    out_type=(jax.ShapeDtypeStruct((NC, NROWS, IN), jnp.float32),
              jax.ShapeDtypeStruct((NC, NS, NROWS), jnp.float32)),
    mesh=_MESH,
    compiler_params=_SC_PARAMS,
    scratch_types=[
        pltpu.VMEM((CHUNK,), jnp.int32),
        pltpu.VMEM((CHUNK,), jnp.int32),
        pltpu.VMEM((CHUNK,), jnp.int32),
        pltpu.VMEM((CHUNK,), jnp.int32),
        pltpu.VMEM((CHUNK, IN), jnp.float32),
        pltpu.VMEM((CHUNK, IN), jnp.float32),
        pltpu.VMEM((NROWS,), jnp.float32),
        pltpu.VMEM_SHARED((NROWS, IN), jnp.float32),
        pltpu.SemaphoreType.DMA,
        pltpu.SemaphoreType.DMA,
        pltpu.SemaphoreType.DMA,
        pltpu.SemaphoreType.DMA,
    ],
)
def _sc_layer1(x_hbm, src_hbm, dst_hbm, zf_hbm, zd_hbm,
               s_out, d_out, src_a, dst_a, src_b, dst_b, rows_a, rows_b,
               deg_v, acc_sh, sem_a, sem_b, isem_a, isem_b):
    _sc_segsum_body(True, x_hbm, src_hbm, dst_hbm, zf_hbm, zd_hbm,
                    s_out, d_out, src_a, dst_a, src_b, dst_b, rows_a, rows_b,
                    deg_v, acc_sh, sem_a, sem_b, isem_a, isem_b)


@functools.partial(
    pl.kernel,
    out_type=jax.ShapeDtypeStruct((NC, NROWS, IN), jnp.float32),
    mesh=_MESH,
    compiler_params=_SC_PARAMS,
    scratch_types=[
        pltpu.VMEM((CHUNK,), jnp.int32),
        pltpu.VMEM((CHUNK,), jnp.int32),
        pltpu.VMEM((CHUNK,), jnp.int32),
        pltpu.VMEM((CHUNK,), jnp.int32),
        pltpu.VMEM((CHUNK, IN), jnp.float32),
        pltpu.VMEM((CHUNK, IN), jnp.float32),
        pltpu.VMEM_SHARED((NROWS, IN), jnp.float32),
        pltpu.SemaphoreType.DMA,
        pltpu.SemaphoreType.DMA,
        pltpu.SemaphoreType.DMA,
        pltpu.SemaphoreType.DMA,
    ],
)
def _sc_layer2(p_hbm, src_hbm, dst_hbm, zf_hbm,
               s_out, src_a, dst_a, src_b, dst_b, rows_a, rows_b,
               acc_sh, sem_a, sem_b, isem_a, isem_b):
    _sc_segsum_body(False, p_hbm, src_hbm, dst_hbm, zf_hbm, None,
                    s_out, None, src_a, dst_a, src_b, dst_b, rows_a, rows_b,
                    None, acc_sh, sem_a, sem_b, isem_a, isem_b)


R = 1024


def _tc1_body(s1_ref, deg_ref, x_ref, w1l_ref, b1_ref, w1r_ref,
              w2l_ref, w2r_ref, pr_ref):
    sv = s1_ref[0] + s1_ref[1]
    deg = jnp.sum(deg_ref[...], axis=(0, 1))[:, None]
    agg = sv / jnp.maximum(deg, 1.0)
    h = jnp.dot(agg, w1l_ref[...], preferred_element_type=jnp.float32)
    h = h + b1_ref[...] + jnp.dot(x_ref[...], w1r_ref[...],
                                  preferred_element_type=jnp.float32)
    h = jnp.maximum(h, 0.0)
    p = jnp.dot(h, w2l_ref[...], preferred_element_type=jnp.float32)
    r = jnp.dot(h, w2r_ref[...], preferred_element_type=jnp.float32)
    pr_ref[...] = jnp.concatenate([p, r], axis=1)


_tc1 = pl.pallas_call(
    _tc1_body,
    grid=(NROWS // R,),
    in_specs=[
        pl.BlockSpec((NC, R, IN), lambda i: (0, i, 0)),
        pl.BlockSpec((NC, NS, R), lambda i: (0, 0, i)),
        pl.BlockSpec((R, IN), lambda i: (i, 0)),
        pl.BlockSpec((IN, HID), lambda i: (0, 0)),
        pl.BlockSpec((1, HID), lambda i: (0, 0)),
        pl.BlockSpec((IN, HID), lambda i: (0, 0)),
        pl.BlockSpec((HID, OUT), lambda i: (0, 0)),
        pl.BlockSpec((HID, OUT), lambda i: (0, 0)),
    ],
    out_specs=pl.BlockSpec((R, IN), lambda i: (i, 0)),
    out_shape=jax.ShapeDtypeStruct((NROWS, IN), jnp.float32),
)


def _tc2_body(s2_ref, deg_ref, pr_ref, b2_ref, o_ref):
    sv = s2_ref[0, :, 0:OUT] + s2_ref[1, :, 0:OUT]
    deg = jnp.sum(deg_ref[...], axis=(0, 1))[:, None]
    o = sv / jnp.maximum(deg, 1.0) + b2_ref[...] + pr_ref[:, OUT:IN]
    m = jnp.max(o, axis=1, keepdims=True)
    e = jnp.exp(o - m)
    o_ref[...] = (o - m) - jnp.log(jnp.sum(e, axis=1, keepdims=True))


_tc2 = pl.pallas_call(
    _tc2_body,
    grid=(NROWS // R,),
    in_specs=[
        pl.BlockSpec((NC, R, IN), lambda i: (0, i, 0)),
        pl.BlockSpec((NC, NS, R), lambda i: (0, 0, i)),
        pl.BlockSpec((R, IN), lambda i: (i, 0)),
        pl.BlockSpec((1, OUT), lambda i: (0, 0)),
    ],
    out_specs=pl.BlockSpec((R, OUT), lambda i: (i, 0)),
    out_shape=jax.ShapeDtypeStruct((NROWS, OUT), jnp.float32),
)


def kernel(x, edge_index, W1l, b1, W1r, W2l, b2, W2r):
    src = jnp.pad(edge_index[0], (0, E_PAD - E))
    dst = jnp.pad(edge_index[1], (0, E_PAD - E), constant_values=N)
    xp = jnp.pad(x, ((0, NROWS - N), (0, 0)))
    zf = jnp.zeros((CHUNK, IN), jnp.float32)
    zd = jnp.zeros((NROWS,), jnp.float32)
    s1, degp = _sc_layer1(xp, src, dst, zf, zd)
    pr = _tc1(s1, degp, xp, W1l.T, b1.reshape(1, HID), W1r.T,
              W2l.T, W2r.T)
    s2 = _sc_layer2(pr, src, dst, zf)
    return _tc2(s2, degp, pr, b2.reshape(1, OUT))[:N]


# speedup vs baseline: 1.1696x; 1.1200x over previous
import dataclasses
import functools

import jax
import jax.numpy as jnp
from jax import lax
from jax.experimental import pallas as pl
from jax.experimental.pallas import tpu as pltpu
from jax.experimental.pallas import tpu_sc as plsc

N = 10000
E = 320000
IN = 128
HID = 256
OUT = 64

NC = 2
NS = 16
L = 16
CHUNK = 128
CPT = 79
EDGES_PER_TILE = CPT * CHUNK
E_PAD = NC * NS * EDGES_PER_TILE
ROWS_PER_TILE = 640
NROWS = NS * ROWS_PER_TILE
ZCOPIES = ROWS_PER_TILE // CHUNK

_MESH = plsc.VectorSubcoreMesh(core_axis_name="c", subcore_axis_name="s")

_SC_PARAMS = pltpu.CompilerParams()
if "needs_layout_passes" in pltpu.CompilerParams.__dataclass_fields__:
    _SC_PARAMS = dataclasses.replace(_SC_PARAMS, needs_layout_passes=False)


def _sc_segsum_body(with_deg, x_hbm, src_hbm, dst_hbm, zf_hbm, zd_hbm,
                    s_out, d_out, src_a, dst_a, src_b, dst_b, rows_a, rows_b,
                    deg_v, acc_sh, sem_a, sem_b, isem_a, isem_b):
    c = lax.axis_index("c")
    s = lax.axis_index("s")
    r0 = s * ROWS_PER_TILE
    pltpu.sync_copy(zf_hbm, rows_a)
    for j in range(ZCOPIES):
        pltpu.sync_copy(rows_a, acc_sh.at[pl.ds(r0 + j * CHUNK, CHUNK)])
    if with_deg:
        pltpu.sync_copy(zd_hbm, deg_v)
    plsc.subcore_barrier()
    base = (c * NS + s) * EDGES_PER_TILE

    def load_idx(j, sbuf, dbuf):
        off = base + j * CHUNK
        pltpu.sync_copy(src_hbm.at[pl.ds(off, CHUNK)], sbuf)
        pltpu.sync_copy(dst_hbm.at[pl.ds(off, CHUNK)], dbuf)

    def issue_idx(j, sbuf, dbuf, isem):
        off = base + j * CHUNK
        pltpu.async_copy(src_hbm.at[pl.ds(off, CHUNK)], sbuf, isem)
        pltpu.async_copy(dst_hbm.at[pl.ds(off, CHUNK)], dbuf, isem)

    def wait_idx(sbuf, dbuf, isem):
        pltpu.make_async_copy(src_hbm.at[pl.ds(0, CHUNK)], sbuf, isem).wait()
        pltpu.make_async_copy(dst_hbm.at[pl.ds(0, CHUNK)], dbuf, isem).wait()

    def deg_update(dbuf):
        if with_deg:
            ones = jnp.ones((L,), jnp.float32)
            for k in range(CHUNK // L):
                plsc.addupdate_scatter(deg_v, [dbuf[pl.ds(k * L, L)]], ones)

    def issue_gather(sbuf, rbuf, sem):
        h = CHUNK // 2
        pltpu.async_copy(x_hbm.at[sbuf.at[pl.ds(0, h)]], rbuf.at[pl.ds(0, h)], sem)
        pltpu.async_copy(x_hbm.at[sbuf.at[pl.ds(h, h)]], rbuf.at[pl.ds(h, h)], sem)

    def wait_gather(sbuf, rbuf, sem):
        pltpu.make_async_copy(x_hbm.at[sbuf], rbuf, sem).wait()

    load_idx(0, src_a, dst_a)
    issue_gather(src_a, rows_a, sem_a)
    issue_idx(1, src_b, dst_b, isem_b)

    def step(k, sbufP, dbufP, rowsP, semP, sbufQ, dbufQ, rowsQ, semQ,
             isemP, isemQ):
        wait_idx(sbufQ, dbufQ, isemQ)
        issue_gather(sbufQ, rowsQ, semQ)
        wait_gather(sbufP, rowsP, semP)
        pltpu.sync_copy(rowsP, acc_sh.at[dbufP], add=True)
        deg_update(dbufP)

        @pl.when(k + 2 < CPT)
        def _():
            issue_idx(k + 2, sbufP, dbufP, isemP)

    def pair(it, carry):
        j = 2 * it
        step(j, src_a, dst_a, rows_a, sem_a, src_b, dst_b, rows_b, sem_b,
             isem_a, isem_b)
        step(j + 1, src_b, dst_b, rows_b, sem_b, src_a, dst_a, rows_a, sem_a,
             isem_b, isem_a)
        return carry

    lax.fori_loop(0, (CPT - 1) // 2, pair, 0)
    wait_gather(src_a, rows_a, sem_a)
    pltpu.sync_copy(rows_a, acc_sh.at[dst_a], add=True)
    deg_update(dst_a)
    plsc.subcore_barrier()
    for j in range(ZCOPIES):
        pltpu.sync_copy(acc_sh.at[pl.ds(r0 + j * CHUNK, CHUNK)], rows_a)
        pltpu.sync_copy(rows_a, s_out.at[c, pl.ds(r0 + j * CHUNK, CHUNK)])
    if with_deg:
        pltpu.sync_copy(deg_v, d_out.at[c, s])


@functools.partial(
    pl.kernel,
    out_type=(jax.ShapeDtypeStruct((NC, NROWS, IN), jnp.float32),
              jax.ShapeDtypeStruct((NC, NS, NROWS), jnp.float32)),
    mesh=_MESH,
    compiler_params=_SC_PARAMS,
    scratch_types=[
        pltpu.VMEM((CHUNK,), jnp.int32),
        pltpu.VMEM((CHUNK,), jnp.int32),
        pltpu.VMEM((CHUNK,), jnp.int32),
        pltpu.VMEM((CHUNK,), jnp.int32),
        pltpu.VMEM((CHUNK, IN), jnp.float32),
        pltpu.VMEM((CHUNK, IN), jnp.float32),
        pltpu.VMEM((NROWS,), jnp.float32),
        pltpu.VMEM_SHARED((NROWS, IN), jnp.float32),
        pltpu.SemaphoreType.DMA,
        pltpu.SemaphoreType.DMA,
        pltpu.SemaphoreType.DMA,
        pltpu.SemaphoreType.DMA,
    ],
)
def _sc_layer1(x_hbm, src_hbm, dst_hbm, zf_hbm, zd_hbm,
               s_out, d_out, src_a, dst_a, src_b, dst_b, rows_a, rows_b,
               deg_v, acc_sh, sem_a, sem_b, isem_a, isem_b):
    _sc_segsum_body(True, x_hbm, src_hbm, dst_hbm, zf_hbm, zd_hbm,
                    s_out, d_out, src_a, dst_a, src_b, dst_b, rows_a, rows_b,
                    deg_v, acc_sh, sem_a, sem_b, isem_a, isem_b)


@functools.partial(
    pl.kernel,
    out_type=jax.ShapeDtypeStruct((NC, NROWS, IN), jnp.float32),
    mesh=_MESH,
    compiler_params=_SC_PARAMS,
    scratch_types=[
        pltpu.VMEM((CHUNK,), jnp.int32),
        pltpu.VMEM((CHUNK,), jnp.int32),
        pltpu.VMEM((CHUNK,), jnp.int32),
        pltpu.VMEM((CHUNK,), jnp.int32),
        pltpu.VMEM((CHUNK, IN), jnp.float32),
        pltpu.VMEM((CHUNK, IN), jnp.float32),
        pltpu.VMEM_SHARED((NROWS, IN), jnp.float32),
        pltpu.SemaphoreType.DMA,
        pltpu.SemaphoreType.DMA,
        pltpu.SemaphoreType.DMA,
        pltpu.SemaphoreType.DMA,
    ],
)
def _sc_layer2(p_hbm, src_hbm, dst_hbm, zf_hbm,
               s_out, src_a, dst_a, src_b, dst_b, rows_a, rows_b,
               acc_sh, sem_a, sem_b, isem_a, isem_b):
    _sc_segsum_body(False, p_hbm, src_hbm, dst_hbm, zf_hbm, None,
                    s_out, None, src_a, dst_a, src_b, dst_b, rows_a, rows_b,
                    None, acc_sh, sem_a, sem_b, isem_a, isem_b)


R = 1024


def _tc1_body(s1_ref, deg_ref, x_ref, w1l_ref, b1_ref, w1r_ref,
              w2l_ref, w2r_ref, pr_ref):
    sv = s1_ref[0] + s1_ref[1]
    deg = jnp.sum(deg_ref[...], axis=(0, 1))[:, None]
    agg = sv / jnp.maximum(deg, 1.0)
    h = jnp.dot(agg, w1l_ref[...], preferred_element_type=jnp.float32)
    h = h + b1_ref[...] + jnp.dot(x_ref[...], w1r_ref[...],
                                  preferred_element_type=jnp.float32)
    h = jnp.maximum(h, 0.0)
    p = jnp.dot(h, w2l_ref[...], preferred_element_type=jnp.float32)
    r = jnp.dot(h, w2r_ref[...], preferred_element_type=jnp.float32)
    pr_ref[...] = jnp.concatenate([p, r], axis=1)


_tc1 = pl.pallas_call(
    _tc1_body,
    grid=(NROWS // R,),
    in_specs=[
        pl.BlockSpec((NC, R, IN), lambda i: (0, i, 0)),
        pl.BlockSpec((NC, NS, R), lambda i: (0, 0, i)),
        pl.BlockSpec((R, IN), lambda i: (i, 0)),
        pl.BlockSpec((IN, HID), lambda i: (0, 0)),
        pl.BlockSpec((1, HID), lambda i: (0, 0)),
        pl.BlockSpec((IN, HID), lambda i: (0, 0)),
        pl.BlockSpec((HID, OUT), lambda i: (0, 0)),
        pl.BlockSpec((HID, OUT), lambda i: (0, 0)),
    ],
    out_specs=pl.BlockSpec((R, IN), lambda i: (i, 0)),
    out_shape=jax.ShapeDtypeStruct((NROWS, IN), jnp.float32),
)


def _tc2_body(s2_ref, deg_ref, pr_ref, b2_ref, o_ref):
    sv = s2_ref[0, :, 0:OUT] + s2_ref[1, :, 0:OUT]
    deg = jnp.sum(deg_ref[...], axis=(0, 1))[:, None]
    o = sv / jnp.maximum(deg, 1.0) + b2_ref[...] + pr_ref[:, OUT:IN]
    m = jnp.max(o, axis=1, keepdims=True)
    e = jnp.exp(o - m)
    o_ref[...] = (o - m) - jnp.log(jnp.sum(e, axis=1, keepdims=True))


_tc2 = pl.pallas_call(
    _tc2_body,
    grid=(NROWS // R,),
    in_specs=[
        pl.BlockSpec((NC, R, IN), lambda i: (0, i, 0)),
        pl.BlockSpec((NC, NS, R), lambda i: (0, 0, i)),
        pl.BlockSpec((R, IN), lambda i: (i, 0)),
        pl.BlockSpec((1, OUT), lambda i: (0, 0)),
    ],
    out_specs=pl.BlockSpec((R, OUT), lambda i: (i, 0)),
    out_shape=jax.ShapeDtypeStruct((NROWS, OUT), jnp.float32),
)


def kernel(x, edge_index, W1l, b1, W1r, W2l, b2, W2r):
    nt = NC * NS
    ppt = EDGES_PER_TILE - E // nt
    src = jnp.pad(edge_index[0].reshape(nt, E // nt),
                  ((0, 0), (0, ppt))).reshape(-1)
    dpad = jnp.broadcast_to(N + jnp.arange(ppt, dtype=jnp.int32), (nt, ppt))
    dst = jnp.concatenate(
        [edge_index[1].reshape(nt, E // nt), dpad], axis=1).reshape(-1)
    xp = jnp.pad(x, ((0, NROWS - N), (0, 0)))
    zf = jnp.zeros((CHUNK, IN), jnp.float32)
    zd = jnp.zeros((NROWS,), jnp.float32)
    s1, degp = _sc_layer1(xp, src, dst, zf, zd)
    pr = _tc1(s1, degp, xp, W1l.T, b1.reshape(1, HID), W1r.T,
              W2l.T, W2r.T)
    s2 = _sc_layer2(pr, src, dst, zf)
    return _tc2(s2, degp, pr, b2.reshape(1, OUT))[:N]


# speedup vs baseline: 1.1724x; 1.0024x over previous
import dataclasses
import functools

import jax
import jax.numpy as jnp
from jax import lax
from jax.experimental import pallas as pl
from jax.experimental.pallas import tpu as pltpu
from jax.experimental.pallas import tpu_sc as plsc

N = 10000
E = 320000
IN = 128
HID = 256
OUT = 64

NC = 2
NS = 16
L = 16
CHUNK = 128
CPT = 79
EDGES_PER_TILE = CPT * CHUNK
E_PAD = NC * NS * EDGES_PER_TILE
ROWS_PER_TILE = 640
NROWS = NS * ROWS_PER_TILE
ZCOPIES = ROWS_PER_TILE // CHUNK

_MESH = plsc.VectorSubcoreMesh(core_axis_name="c", subcore_axis_name="s")

_SC_PARAMS = pltpu.CompilerParams()
if "needs_layout_passes" in pltpu.CompilerParams.__dataclass_fields__:
    _SC_PARAMS = dataclasses.replace(_SC_PARAMS, needs_layout_passes=False)


def _sc_segsum_body(with_deg, x_hbm, src_hbm, dst_hbm, zf_hbm, zd_hbm,
                    s_out, d_out, src_a, dst_a, src_b, dst_b, rows_a, rows_b,
                    deg_v, acc_sh, sem_a, sem_b, isem_a, isem_b):
    c = lax.axis_index("c")
    s = lax.axis_index("s")
    r0 = s * ROWS_PER_TILE
    pltpu.sync_copy(zf_hbm, rows_a)
    for j in range(ZCOPIES):
        pltpu.sync_copy(rows_a, acc_sh.at[pl.ds(r0 + j * CHUNK, CHUNK)])
    if with_deg:
        pltpu.sync_copy(zd_hbm, deg_v)
    plsc.subcore_barrier()
    base = (c * NS + s) * EDGES_PER_TILE

    def load_idx(j, sbuf, dbuf):
        off = base + j * CHUNK
        pltpu.sync_copy(src_hbm.at[pl.ds(off, CHUNK)], sbuf)
        pltpu.sync_copy(dst_hbm.at[pl.ds(off, CHUNK)], dbuf)

    def issue_idx(j, sbuf, dbuf, isem):
        off = base + j * CHUNK
        pltpu.async_copy(src_hbm.at[pl.ds(off, CHUNK)], sbuf, isem)
        pltpu.async_copy(dst_hbm.at[pl.ds(off, CHUNK)], dbuf, isem)

    def wait_idx(sbuf, dbuf, isem):
        pltpu.make_async_copy(src_hbm.at[pl.ds(0, CHUNK)], sbuf, isem).wait()
        pltpu.make_async_copy(dst_hbm.at[pl.ds(0, CHUNK)], dbuf, isem).wait()

    def deg_update(dbuf):
        if with_deg:
            ones = jnp.ones((L,), jnp.float32)
            for k in range(CHUNK // L):
                plsc.addupdate_scatter(deg_v, [dbuf[pl.ds(k * L, L)]], ones)

    def issue_gather(sbuf, rbuf, sem):
        h = CHUNK // 2
        pltpu.async_copy(x_hbm.at[sbuf.at[pl.ds(0, h)]], rbuf.at[pl.ds(0, h)], sem)
        pltpu.async_copy(x_hbm.at[sbuf.at[pl.ds(h, h)]], rbuf.at[pl.ds(h, h)], sem)

    def wait_gather(sbuf, rbuf, sem):
        pltpu.make_async_copy(x_hbm.at[sbuf], rbuf, sem).wait()

    load_idx(0, src_a, dst_a)
    issue_gather(src_a, rows_a, sem_a)
    issue_idx(1, src_b, dst_b, isem_b)

    def step(k, sbufP, dbufP, rowsP, semP, sbufQ, dbufQ, rowsQ, semQ,
             isemP, isemQ):
        wait_idx(sbufQ, dbufQ, isemQ)
        issue_gather(sbufQ, rowsQ, semQ)
        wait_gather(sbufP, rowsP, semP)
        pltpu.sync_copy(rowsP, acc_sh.at[dbufP], add=True)
        deg_update(dbufP)

        @pl.when(k + 2 < CPT)
        def _():
            issue_idx(k + 2, sbufP, dbufP, isemP)

    def pair(it, carry):
        j = 2 * it
        step(j, src_a, dst_a, rows_a, sem_a, src_b, dst_b, rows_b, sem_b,
             isem_a, isem_b)
        step(j + 1, src_b, dst_b, rows_b, sem_b, src_a, dst_a, rows_a, sem_a,
             isem_b, isem_a)
        return carry

    lax.fori_loop(0, (CPT - 1) // 2, pair, 0)
    wait_gather(src_a, rows_a, sem_a)
    pltpu.sync_copy(rows_a, acc_sh.at[dst_a], add=True)
    deg_update(dst_a)
    plsc.subcore_barrier()
    for j in range(ZCOPIES):
        pltpu.sync_copy(acc_sh.at[pl.ds(r0 + j * CHUNK, CHUNK)], rows_a)
        pltpu.sync_copy(rows_a, s_out.at[c, pl.ds(r0 + j * CHUNK, CHUNK)])
    if with_deg:
        pltpu.sync_copy(deg_v, d_out.at[c, s])


@functools.partial(
    pl.kernel,
    out_type=(jax.ShapeDtypeStruct((NC, NROWS, IN), jnp.float32),
              jax.ShapeDtypeStruct((NC, NS, NROWS), jnp.float32)),
    mesh=_MESH,
    compiler_params=_SC_PARAMS,
    scratch_types=[
        pltpu.VMEM((CHUNK,), jnp.int32),
        pltpu.VMEM((CHUNK,), jnp.int32),
        pltpu.VMEM((CHUNK,), jnp.int32),
        pltpu.VMEM((CHUNK,), jnp.int32),
        pltpu.VMEM((CHUNK, IN), jnp.float32),
        pltpu.VMEM((CHUNK, IN), jnp.float32),
        pltpu.VMEM((NROWS,), jnp.float32),
        pltpu.VMEM_SHARED((NROWS, IN), jnp.float32),
        pltpu.SemaphoreType.DMA,
        pltpu.SemaphoreType.DMA,
        pltpu.SemaphoreType.DMA,
        pltpu.SemaphoreType.DMA,
    ],
)
def _sc_layer1(x_hbm, src_hbm, dst_hbm, zf_hbm, zd_hbm,
               s_out, d_out, src_a, dst_a, src_b, dst_b, rows_a, rows_b,
               deg_v, acc_sh, sem_a, sem_b, isem_a, isem_b):
    _sc_segsum_body(True, x_hbm, src_hbm, dst_hbm, zf_hbm, zd_hbm,
                    s_out, d_out, src_a, dst_a, src_b, dst_b, rows_a, rows_b,
                    deg_v, acc_sh, sem_a, sem_b, isem_a, isem_b)


@functools.partial(
    pl.kernel,
    out_type=jax.ShapeDtypeStruct((NC, NROWS, IN), jnp.float32),
    mesh=_MESH,
    compiler_params=_SC_PARAMS,
    scratch_types=[
        pltpu.VMEM((CHUNK,), jnp.int32),
        pltpu.VMEM((CHUNK,), jnp.int32),
        pltpu.VMEM((CHUNK,), jnp.int32),
        pltpu.VMEM((CHUNK,), jnp.int32),
        pltpu.VMEM((CHUNK, IN), jnp.float32),
        pltpu.VMEM((CHUNK, IN), jnp.float32),
        pltpu.VMEM_SHARED((NROWS, IN), jnp.float32),
        pltpu.SemaphoreType.DMA,
        pltpu.SemaphoreType.DMA,
        pltpu.SemaphoreType.DMA,
        pltpu.SemaphoreType.DMA,
    ],
)
def _sc_layer2(p_hbm, src_hbm, dst_hbm, zf_hbm,
               s_out, src_a, dst_a, src_b, dst_b, rows_a, rows_b,
               acc_sh, sem_a, sem_b, isem_a, isem_b):
    _sc_segsum_body(False, p_hbm, src_hbm, dst_hbm, zf_hbm, None,
                    s_out, None, src_a, dst_a, src_b, dst_b, rows_a, rows_b,
                    None, acc_sh, sem_a, sem_b, isem_a, isem_b)


R = 1024


def _tc1a_body(x_ref, w1r_ref, b1_ref, xr_ref):
    xr_ref[...] = b1_ref[...] + jnp.dot(x_ref[...], w1r_ref[...],
                                        preferred_element_type=jnp.float32)


_tc1a = pl.pallas_call(
    _tc1a_body,
    grid=(NROWS // R,),
    in_specs=[
        pl.BlockSpec((R, IN), lambda i: (i, 0)),
        pl.BlockSpec((IN, HID), lambda i: (0, 0)),
        pl.BlockSpec((1, HID), lambda i: (0, 0)),
    ],
    out_specs=pl.BlockSpec((R, HID), lambda i: (i, 0)),
    out_shape=jax.ShapeDtypeStruct((NROWS, HID), jnp.float32),
)


def _tc1_body(s1_ref, deg_ref, xr_ref, w1l_ref, w2l_ref, w2r_ref, pr_ref):
    sv = s1_ref[0] + s1_ref[1]
    deg = jnp.sum(deg_ref[...], axis=(0, 1))[:, None]
    agg = sv / jnp.maximum(deg, 1.0)
    h = jnp.dot(agg, w1l_ref[...], preferred_element_type=jnp.float32)
    h = jnp.maximum(h + xr_ref[...], 0.0)
    p = jnp.dot(h, w2l_ref[...], preferred_element_type=jnp.float32)
    r = jnp.dot(h, w2r_ref[...], preferred_element_type=jnp.float32)
    pr_ref[...] = jnp.concatenate([p, r], axis=1)


_tc1 = pl.pallas_call(
    _tc1_body,
    grid=(NROWS // R,),
    in_specs=[
        pl.BlockSpec((NC, R, IN), lambda i: (0, i, 0)),
        pl.BlockSpec((NC, NS, R), lambda i: (0, 0, i)),
        pl.BlockSpec((R, HID), lambda i: (i, 0)),
        pl.BlockSpec((IN, HID), lambda i: (0, 0)),
        pl.BlockSpec((HID, OUT), lambda i: (0, 0)),
        pl.BlockSpec((HID, OUT), lambda i: (0, 0)),
    ],
    out_specs=pl.BlockSpec((R, IN), lambda i: (i, 0)),
    out_shape=jax.ShapeDtypeStruct((NROWS, IN), jnp.float32),
)


def _tc2_body(s2_ref, deg_ref, pr_ref, b2_ref, o_ref):
    sv = s2_ref[0, :, 0:OUT] + s2_ref[1, :, 0:OUT]
    deg = jnp.sum(deg_ref[...], axis=(0, 1))[:, None]
    o = sv / jnp.maximum(deg, 1.0) + b2_ref[...] + pr_ref[:, OUT:IN]
    m = jnp.max(o, axis=1, keepdims=True)
    e = jnp.exp(o - m)
    o_ref[...] = (o - m) - jnp.log(jnp.sum(e, axis=1, keepdims=True))


_tc2 = pl.pallas_call(
    _tc2_body,
    grid=(NROWS // R,),
    in_specs=[
        pl.BlockSpec((NC, R, IN), lambda i: (0, i, 0)),
        pl.BlockSpec((NC, NS, R), lambda i: (0, 0, i)),
        pl.BlockSpec((R, IN), lambda i: (i, 0)),
        pl.BlockSpec((1, OUT), lambda i: (0, 0)),
    ],
    out_specs=pl.BlockSpec((R, OUT), lambda i: (i, 0)),
    out_shape=jax.ShapeDtypeStruct((NROWS, OUT), jnp.float32),
)


def kernel(x, edge_index, W1l, b1, W1r, W2l, b2, W2r):
    nt = NC * NS
    ppt = EDGES_PER_TILE - E // nt
    src = jnp.pad(edge_index[0].reshape(nt, E // nt),
                  ((0, 0), (0, ppt))).reshape(-1)
    dpad = jnp.broadcast_to(N + jnp.arange(ppt, dtype=jnp.int32), (nt, ppt))
    dst = jnp.concatenate(
        [edge_index[1].reshape(nt, E // nt), dpad], axis=1).reshape(-1)
    xp = jnp.pad(x, ((0, NROWS - N), (0, 0)))
    zf = jnp.zeros((CHUNK, IN), jnp.float32)
    zd = jnp.zeros((NROWS,), jnp.float32)
    xr = _tc1a(xp, W1r.T, b1.reshape(1, HID))
    s1, degp = _sc_layer1(xp, src, dst, zf, zd)
    pr = _tc1(s1, degp, xr, W1l.T, W2l.T, W2r.T)
    s2 = _sc_layer2(pr, src, dst, zf)
    return _tc2(s2, degp, pr, b2.reshape(1, OUT))[:N]
